# baseline (speedup 1.0000x reference)
import functools

import jax
import jax.numpy as jnp
from jax.experimental import pallas as pl
from jax.experimental.pallas import tpu as pltpu

_E = 8
_GS = 2


def _ffn_body(tok_ref, g_ref, wup_ref, bup_ref, wdn_ref, bdn_ref, out_ref):
    tok = tok_ref[0].astype(jnp.bfloat16)
    wu = wup_ref[0].astype(jnp.bfloat16)
    up = jnp.dot(tok, wu, preferred_element_type=jnp.float32)
    up = up + bup_ref[0]
    a = up * jax.nn.sigmoid(up)
    wd = wdn_ref[0].astype(jnp.bfloat16)
    dn = jnp.dot(a.astype(jnp.bfloat16), wd, preferred_element_type=jnp.float32)
    dn = dn + bdn_ref[0]
    g = g_ref[0].T
    out_ref[0] = (dn * g).astype(jnp.bfloat16)


def kernel(x, routing_logits, batch_size, seq_len, W_up, b_up, W_down, b_down):
    bs, hidden = x.shape
    ff = W_up.shape[-1]
    k = bs // _E

    S = jax.nn.softmax(routing_logits, axis=-1)
    G_t, idx_t = jax.lax.top_k(S.T, k)
    tokens = jnp.take(x, idx_t, axis=0)

    weighted = pl.pallas_call(
        _ffn_body,
        grid=(_E,),
        in_specs=[
            pl.BlockSpec((1, k, hidden), lambda e: (e, 0, 0)),
            pl.BlockSpec((1, 1, k), lambda e: (e, 0, 0)),
            pl.BlockSpec((1, hidden, ff), lambda e: (e, 0, 0)),
            pl.BlockSpec((1, 1, ff), lambda e: (e, 0, 0)),
            pl.BlockSpec((1, ff, hidden), lambda e: (e // _GS, 0, 0)),
            pl.BlockSpec((1, 1, hidden), lambda e: (e // _GS, 0, 0)),
        ],
        out_specs=pl.BlockSpec((1, k, hidden), lambda e: (e, 0, 0)),
        out_shape=jax.ShapeDtypeStruct((_E, k, hidden), jnp.bfloat16),
    )(tokens, G_t[:, None, :], W_up, b_up[:, None, :], W_down, b_down[:, None, :])

    y = jnp.zeros((bs, hidden), x.dtype).at[idx_t.reshape(-1)].add(
        weighted.reshape(-1, hidden).astype(jnp.float32)
    )
    return y


# speedup vs baseline: 1.0164x; 1.0164x over previous
import functools

import jax
import jax.numpy as jnp
from jax.experimental import pallas as pl
from jax.experimental.pallas import tpu as pltpu

_E = 8
_GS = 2


def _ffn_body(tok_ref, g_ref, wup_ref, bup_ref, wdn_ref, bdn_ref, out_ref,
              acc_scr):
    j = pl.program_id(1)
    tok = tok_ref[0].astype(jnp.bfloat16)
    wu = wup_ref[0].astype(jnp.bfloat16)
    part = jnp.dot(tok, wu, preferred_element_type=jnp.float32)

    @pl.when(j == 0)
    def _():
        acc_scr[...] = part

    @pl.when(j == 1)
    def _():
        up = acc_scr[...] + part + bup_ref[0]
        a = up * jax.nn.sigmoid(up)
        wd = wdn_ref[0].astype(jnp.bfloat16)
        dn = jnp.dot(a.astype(jnp.bfloat16), wd,
                     preferred_element_type=jnp.float32)
        out_ref[0] = (dn + bdn_ref[0]) * g_ref[0].T


def kernel(x, routing_logits, batch_size, seq_len, W_up, b_up, W_down, b_down):
    bs, hidden = x.shape
    ff = W_up.shape[-1]
    k = bs // _E
    hh = hidden // 2

    S = jax.nn.softmax(routing_logits, axis=-1)
    G_t, idx_t = jax.lax.top_k(S.T, k)
    tokens = jnp.take(x, idx_t, axis=0)

    weighted = pl.pallas_call(
        _ffn_body,
        grid=(_E, 2),
        in_specs=[
            pl.BlockSpec((1, k, hh), lambda e, j: (e, 0, j)),
            pl.BlockSpec((1, 1, k), lambda e, j: (e, 0, 0)),
            pl.BlockSpec((1, hh, ff), lambda e, j: (e, j, 0)),
            pl.BlockSpec((1, 1, ff), lambda e, j: (e, 0, 0)),
            pl.BlockSpec((1, ff, hidden), lambda e, j: (e // _GS, 0, 0)),
            pl.BlockSpec((1, 1, hidden), lambda e, j: (e // _GS, 0, 0)),
        ],
        out_specs=pl.BlockSpec((1, k, hidden), lambda e, j: (e, 0, 0)),
        out_shape=jax.ShapeDtypeStruct((_E, k, hidden), jnp.float32),
        scratch_shapes=[pltpu.VMEM((k, ff), jnp.float32)],
        compiler_params=pltpu.CompilerParams(
            dimension_semantics=("arbitrary", "arbitrary"),
        ),
    )(tokens, G_t[:, None, :], W_up, b_up[:, None, :], W_down,
      b_down[:, None, :])

    y = jnp.zeros((bs, hidden), x.dtype).at[idx_t.reshape(-1)].add(
        weighted.reshape(-1, hidden)
    )
    return y


# speedup vs baseline: 1.0380x; 1.0213x over previous
import functools

import jax
import jax.numpy as jnp
from jax.experimental import pallas as pl
from jax.experimental.pallas import tpu as pltpu

_E = 8
_GS = 2


def _ffn_body(tok_ref, g_ref, wup_ref, bup_ref, wdn_ref, bdn_ref, out_ref):
    up = jnp.dot(tok_ref[0], wup_ref[0],
                 preferred_element_type=jnp.float32)
    up = up + bup_ref[0]
    a = up * jax.nn.sigmoid(up)
    dn = jnp.dot(a, wdn_ref[0], preferred_element_type=jnp.float32)
    dn = dn + bdn_ref[0]
    out_ref[0] = dn * g_ref[0].T


def kernel(x, routing_logits, batch_size, seq_len, W_up, b_up, W_down, b_down):
    bs, hidden = x.shape
    ff = W_up.shape[-1]
    k = bs // _E

    S = jax.nn.softmax(routing_logits, axis=-1)
    G_t, idx_t = jax.lax.top_k(S.T, k)
    tokens = jnp.take(x, idx_t, axis=0)

    weighted = pl.pallas_call(
        _ffn_body,
        grid=(_E,),
        in_specs=[
            pl.BlockSpec((1, k, hidden), lambda e: (e, 0, 0)),
            pl.BlockSpec((1, 1, k), lambda e: (e, 0, 0)),
            pl.BlockSpec((1, hidden, ff), lambda e: (e, 0, 0)),
            pl.BlockSpec((1, 1, ff), lambda e: (e, 0, 0)),
            pl.BlockSpec((1, ff, hidden), lambda e: (e // _GS, 0, 0)),
            pl.BlockSpec((1, 1, hidden), lambda e: (e // _GS, 0, 0)),
        ],
        out_specs=pl.BlockSpec((1, k, hidden), lambda e: (e, 0, 0)),
        out_shape=jax.ShapeDtypeStruct((_E, k, hidden), jnp.float32),
    )(tokens, G_t[:, None, :], W_up, b_up[:, None, :], W_down, b_down[:, None, :])

    y = jnp.zeros((bs, hidden), x.dtype).at[idx_t.reshape(-1)].add(
        weighted.reshape(-1, hidden)
    )
    return y
